# baseline (speedup 1.0000x reference)
import math

import jax
import jax.numpy as jnp
from jax.experimental import pallas as pl

_N_ATOMS = 200
_BATCH = 16
_BINS = 32
_F = 64
_R = 200
_C = 256
_TILES = _N_ATOMS // _R
_STEP = 2.0 / (_BINS - 1)
_NEG = -1e30

_A = (1.5707288, -0.2121144, 0.0742610, -0.0187293)


def _asin(t):
    s = jnp.abs(t)
    p = ((_A[3] * s + _A[2]) * s + _A[1]) * s + _A[0]
    v = 0.5 * math.pi - jnp.sqrt(jnp.maximum(1.0 - s, 0.0)) * p
    tb = jax.lax.bitcast_convert_type(t, jnp.int32)
    vb = jax.lax.bitcast_convert_type(v, jnp.int32)
    return jax.lax.bitcast_convert_type(
        vb | (tb & jnp.int32(-2147483648)), jnp.float32)


def _lrelu(x):
    return jnp.maximum(x, 0.01 * x)


def _cross(ax, ay, az, bx, by, bz):
    return ay * bz - az * by, az * bx - ax * bz, ax * by - ay * bx


def _body(rowc_ref, colc_ref, nf_t_ref, nf_f_ref, basis_ref, wq_ref,
          wk_ref, wv_ref, out_ref):
    rt = pl.program_id(1)

    rc = rowc_ref[0]
    cc = colc_ref[0]
    rx, ry, rz = rc[:, 0:1], rc[:, 1:2], rc[:, 2:3]
    rxn, ryn, rzn = rc[:, 3:4], rc[:, 4:5], rc[:, 5:6]
    cx, cy, cz = (jnp.broadcast_to(cc[i:i + 1, :], (_R, _C))
                  for i in range(3))
    cxn, cyn, czn = (jnp.broadcast_to(cc[i:i + 1, :], (_R, _C))
                     for i in range(3, 6))

    d0x, d0y, d0z = cx - rx, cy - ry, cz - rz
    d1x, d1y, d1z = cxn - rx, cyn - ry, czn - rz
    d2x, d2y, d2z = cx - rxn, cy - ryn, cz - rzn
    d3x, d3y, d3z = cxn - rxn, cyn - ryn, czn - rzn

    c0 = _cross(d0x, d0y, d0z, d1x, d1y, d1z)
    c1 = _cross(d1x, d1y, d1z, d3x, d3y, d3z)
    c2 = _cross(d3x, d3y, d3z, d2x, d2y, d2z)
    c3 = _cross(d2x, d2y, d2z, d0x, d0y, d0z)
    n0, n1, n2, n3 = (jnp.maximum(x * x + y * y + z * z, 1e-30)
                      for (x, y, z) in (c0, c1, c2, c3))

    dots = jnp.concatenate(
        [(ax * bx + ay * by + az * bz) * jax.lax.rsqrt(na * nb)
         for (ax, ay, az), (bx, by, bz), na, nb in
         ((c0, c1, n0, n1), (c1, c2, n1, n2), (c2, c3, n2, n3),
          (c3, c0, n3, n0))], axis=0)
    asins = _asin(dots)
    omega = (asins[:_R] + asins[_R:2 * _R] + asins[2 * _R:3 * _R]
             + asins[3 * _R:])

    ex, ey, ez = cxn - cx, cyn - cy, czn - cz
    fx, fy, fz = rxn - rx, ryn - ry, rzn - rz
    gx, gy, gz = _cross(ex, ey, ez, fx, fy, fz)
    sraw = gx * d0x + gy * d0y + gz * d0z
    sgn = jnp.where(sraw > 0.0, 1.0, jnp.where(sraw < 0.0, -1.0, 0.0))

    w = omega * sgn * (1.0 / (2.0 * math.pi))

    d_id = rt * _R + jax.lax.broadcasted_iota(jnp.int32, (_R, _C), 0)
    u_id = jax.lax.broadcasted_iota(jnp.int32, (_R, _C), 1)
    valid = ((jnp.abs(u_id - d_id) >= 2) & (u_id <= _N_ATOMS - 2)
             & (d_id <= _N_ATOMS - 2))
    w = jnp.where(valid, w, 0.0)

    qt = _lrelu(jnp.dot(nf_t_ref[...], wq_ref[...]))
    kf = _lrelu(jnp.dot(nf_f_ref[...], wk_ref[...]))
    logits = jax.lax.dot_general(
        qt, kf, (((1,), (1,)), ((), ()))) * (math.log2(math.e)
                                             / math.sqrt(_F))
    logits = jnp.concatenate(
        [logits, jnp.full((_R, _C - _N_ATOMS), _NEG, jnp.float32)], axis=1)
    logits = jnp.where(valid, logits, _NEG)
    m = jnp.max(logits, axis=1, keepdims=True)
    lm = logits - m
    denom = jnp.sum(jnp.exp2(lm), axis=1, keepdims=True)
    la2 = lm - jnp.log2(denom)

    bwt = jax.lax.dot_general(
        wv_ref[...], basis_ref[...],
        (((0,), (1,)), ((), ()))) * (1.0 / 1.12)
    sql = math.sqrt(math.log2(math.e))
    cs = jax.lax.broadcasted_iota(
        jnp.int32, (1, _BINS, 1), 1).astype(jnp.float32) * sql - (sql / _STEP)
    ws = w * (sql / _STEP)
    diff = ws[:, None, :] - cs
    soft = jnp.exp2(la2[:, None, :] - diff * diff)
    z3 = jnp.stack(
        [jax.lax.dot_general(bwt, soft[r], (((1,), (0,)), ((), ())))
         for r in range(_R)], axis=0)
    g = jnp.sum(soft, axis=2)
    msg_lin = jax.lax.dot_general(g, bwt, (((1,), (1,)), ((), ())))
    msg = 0.505 * msg_lin + 0.495 * jnp.sum(jnp.abs(z3), axis=2)
    row_ok = (rt * _R + jax.lax.broadcasted_iota(jnp.int32, (_R, 1), 0)
              ) <= _N_ATOMS - 2
    out_ref[...] = nf_t_ref[...] + jnp.where(row_ok, msg, 0.0)


def kernel(node_features, xyz, basis, Wq, Wk, Wv):
    B, N, C = _BATCH, _N_ATOMS, _C
    xyz_r = xyz.reshape(B, N, 3)
    xyz_n = jnp.concatenate([xyz_r[:, 1:], xyz_r[:, -1:]], axis=1)

    rowc = jnp.concatenate(
        [xyz_r, xyz_n, jnp.zeros((B, N, 2), jnp.float32)], axis=-1)
    colc = jnp.concatenate(
        [jnp.transpose(xyz_r, (0, 2, 1)), jnp.transpose(xyz_n, (0, 2, 1)),
         jnp.zeros((B, 2, N), jnp.float32)], axis=1)
    colc = jnp.pad(colc, ((0, 0), (0, 0), (0, C - N)))

    grid = (B, _TILES)
    out = pl.pallas_call(
        _body,
        grid=grid,
        in_specs=[
            pl.BlockSpec((1, _R, 8), lambda b, t: (b, t, 0)),
            pl.BlockSpec((1, 8, C), lambda b, t: (b, 0, 0)),
            pl.BlockSpec((_R, _F), lambda b, t: (b * _TILES + t, 0)),
            pl.BlockSpec((N, _F), lambda b, t: (b, 0)),
            pl.BlockSpec((_BINS, _F), lambda b, t: (0, 0)),
            pl.BlockSpec((_F, _F), lambda b, t: (0, 0)),
            pl.BlockSpec((_F, _F), lambda b, t: (0, 0)),
            pl.BlockSpec((_F, _F), lambda b, t: (0, 0)),
        ],
        out_specs=pl.BlockSpec((_R, _F), lambda b, t: (b * _TILES + t, 0)),
        out_shape=jax.ShapeDtypeStruct((B * N, _F), jnp.float32),
    )(rowc, colc, node_features, node_features, basis, Wq, Wk, Wv)
    return out


# speedup vs baseline: 1.0224x; 1.0224x over previous
import math

import jax
import jax.numpy as jnp
from jax.experimental import pallas as pl

_N_ATOMS = 200
_BATCH = 16
_BINS = 32
_F = 64
_R = 200
_C = 256
_TILES = _N_ATOMS // _R
_STEP = 2.0 / (_BINS - 1)
_NEG = -1e30

_A = (1.5707288, -0.2121144, 0.0742610, -0.0187293)


def _asin(t):
    s = jnp.abs(t)
    p = ((_A[3] * s + _A[2]) * s + _A[1]) * s + _A[0]
    v = 0.5 * math.pi - jnp.sqrt(jnp.maximum(1.0 - s, 0.0)) * p
    tb = jax.lax.bitcast_convert_type(t, jnp.int32)
    vb = jax.lax.bitcast_convert_type(v, jnp.int32)
    return jax.lax.bitcast_convert_type(
        vb | (tb & jnp.int32(-2147483648)), jnp.float32)


def _lrelu(x):
    return jnp.maximum(x, 0.01 * x)


def _cross(ax, ay, az, bx, by, bz):
    return ay * bz - az * by, az * bx - ax * bz, ax * by - ay * bx


def _body(rowc_ref, colc_ref, nf_t_ref, nf_f_ref, basis_ref, wq_ref,
          wk_ref, wv_ref, out_ref):
    rt = pl.program_id(1)

    rc = rowc_ref[0]
    cc = colc_ref[0]
    rx, ry, rz = rc[:, 0:1], rc[:, 1:2], rc[:, 2:3]
    rxn, ryn, rzn = rc[:, 3:4], rc[:, 4:5], rc[:, 5:6]
    cx, cy, cz = (jnp.broadcast_to(cc[i:i + 1, :], (_R, _C))
                  for i in range(3))
    cxn, cyn, czn = (jnp.broadcast_to(cc[i:i + 1, :], (_R, _C))
                     for i in range(3, 6))

    d0x, d0y, d0z = cx - rx, cy - ry, cz - rz
    d1x, d1y, d1z = cxn - rx, cyn - ry, czn - rz
    d2x, d2y, d2z = cx - rxn, cy - ryn, cz - rzn
    d3x, d3y, d3z = cxn - rxn, cyn - ryn, czn - rzn

    c0 = _cross(d0x, d0y, d0z, d1x, d1y, d1z)
    c1 = _cross(d1x, d1y, d1z, d3x, d3y, d3z)
    c2 = _cross(d3x, d3y, d3z, d2x, d2y, d2z)
    c3 = _cross(d2x, d2y, d2z, d0x, d0y, d0z)
    n0, n1, n2, n3 = (jnp.maximum(x * x + y * y + z * z, 1e-30)
                      for (x, y, z) in (c0, c1, c2, c3))

    dots = jnp.concatenate(
        [(ax * bx + ay * by + az * bz) * jax.lax.rsqrt(na * nb)
         for (ax, ay, az), (bx, by, bz), na, nb in
         ((c0, c1, n0, n1), (c1, c2, n1, n2), (c2, c3, n2, n3),
          (c3, c0, n3, n0))], axis=0)
    asins = _asin(dots)
    omega = (asins[:_R] + asins[_R:2 * _R] + asins[2 * _R:3 * _R]
             + asins[3 * _R:])

    ex, ey, ez = cxn - cx, cyn - cy, czn - cz
    fx, fy, fz = rxn - rx, ryn - ry, rzn - rz
    gx, gy, gz = _cross(ex, ey, ez, fx, fy, fz)
    sraw = gx * d0x + gy * d0y + gz * d0z
    sgn = jnp.where(sraw > 0.0, 1.0, jnp.where(sraw < 0.0, -1.0, 0.0))

    w = omega * sgn * (1.0 / (2.0 * math.pi))

    d_id = rt * _R + jax.lax.broadcasted_iota(jnp.int32, (_R, _C), 0)
    u_id = jax.lax.broadcasted_iota(jnp.int32, (_R, _C), 1)
    valid = ((jnp.abs(u_id - d_id) >= 2) & (u_id <= _N_ATOMS - 2)
             & (d_id <= _N_ATOMS - 2))
    w = jnp.where(valid, w, 0.0)

    qt = _lrelu(jnp.dot(nf_t_ref[...], wq_ref[...]))
    kf = _lrelu(jnp.dot(nf_f_ref[...], wk_ref[...]))
    logits = jax.lax.dot_general(
        qt, kf, (((1,), (1,)), ((), ()))) * (math.log2(math.e)
                                             / math.sqrt(_F))
    logits = jnp.concatenate(
        [logits, jnp.full((_R, _C - _N_ATOMS), _NEG, jnp.float32)], axis=1)
    logits = jnp.where(valid, logits, _NEG)
    m = jnp.max(logits, axis=1, keepdims=True)
    e = jnp.exp2(logits - m)
    attn = e / jnp.sum(e, axis=1, keepdims=True)

    bwt = jax.lax.dot_general(
        wv_ref[...], basis_ref[...],
        (((0,), (1,)), ((), ()))) * (1.0 / 1.12)
    sql = math.sqrt(math.log2(math.e))
    cs = jax.lax.broadcasted_iota(
        jnp.int32, (1, _BINS, 1), 1).astype(jnp.float32) * sql - (sql / _STEP)
    ws = w * (sql / _STEP)
    la = jnp.log2(attn)
    diff = ws[:, None, :] - cs
    soft = jnp.exp2(la[:, None, :] - diff * diff)
    z3 = jnp.stack(
        [jax.lax.dot_general(bwt, soft[r], (((1,), (0,)), ((), ())))
         for r in range(_R)], axis=0)
    g = jnp.sum(soft, axis=2)
    msg_lin = jax.lax.dot_general(g, bwt, (((1,), (1,)), ((), ())))
    msg = 0.505 * msg_lin + 0.495 * jnp.sum(jnp.abs(z3), axis=2)
    row_ok = (rt * _R + jax.lax.broadcasted_iota(jnp.int32, (_R, 1), 0)
              ) <= _N_ATOMS - 2
    out_ref[...] = nf_t_ref[...] + jnp.where(row_ok, msg, 0.0)


def kernel(node_features, xyz, basis, Wq, Wk, Wv):
    B, N, C = _BATCH, _N_ATOMS, _C
    xyz_r = xyz.reshape(B, N, 3)
    xyz_n = jnp.concatenate([xyz_r[:, 1:], xyz_r[:, -1:]], axis=1)

    rowc = jnp.concatenate(
        [xyz_r, xyz_n, jnp.zeros((B, N, 2), jnp.float32)], axis=-1)
    colc = jnp.concatenate(
        [jnp.transpose(xyz_r, (0, 2, 1)), jnp.transpose(xyz_n, (0, 2, 1)),
         jnp.zeros((B, 2, N), jnp.float32)], axis=1)
    colc = jnp.pad(colc, ((0, 0), (0, 0), (0, C - N)))

    grid = (B, _TILES)
    out = pl.pallas_call(
        _body,
        grid=grid,
        in_specs=[
            pl.BlockSpec((1, _R, 8), lambda b, t: (b, t, 0)),
            pl.BlockSpec((1, 8, C), lambda b, t: (b, 0, 0)),
            pl.BlockSpec((_R, _F), lambda b, t: (b * _TILES + t, 0)),
            pl.BlockSpec((N, _F), lambda b, t: (b, 0)),
            pl.BlockSpec((_BINS, _F), lambda b, t: (0, 0)),
            pl.BlockSpec((_F, _F), lambda b, t: (0, 0)),
            pl.BlockSpec((_F, _F), lambda b, t: (0, 0)),
            pl.BlockSpec((_F, _F), lambda b, t: (0, 0)),
        ],
        out_specs=pl.BlockSpec((_R, _F), lambda b, t: (b * _TILES + t, 0)),
        out_shape=jax.ShapeDtypeStruct((B * N, _F), jnp.float32),
    )(rowc, colc, node_features, node_features, basis, Wq, Wk, Wv)
    return out
